# baseline (speedup 1.0000x reference)
import functools

import jax
import jax.numpy as jnp
from jax import lax
from jax.experimental import pallas as pl
from jax.experimental.pallas import tpu as pltpu
from jax.experimental.pallas import tpu_sc as plsc

NC, NS = 2, 16
NW = NC * NS
CHUNK = 200

BM_NODE = 1000
BM_EDGE = 2000


def _row_spec(bm, k):
    return pl.BlockSpec((bm, k), lambda i: (i, 0))


def _full_spec(shape):
    return pl.BlockSpec(shape, lambda i: tuple(0 for _ in shape))


def _relu(x):
    return jnp.maximum(x, 0.0)


def _dot(x, w):
    return jnp.dot(x, w, preferred_element_type=jnp.float32)


def _bdot(x, w):
    return jnp.dot(x.astype(jnp.bfloat16), w.astype(jnp.bfloat16),
                   preferred_element_type=jnp.float32)


def _mlp3_body(x_ref, w1, b1, w2, b2, w3, b3, o_ref):
    h = _relu(_dot(x_ref[...], w1[...]) + b1[...])
    h = _relu(_dot(h, w2[...]) + b2[...])
    o_ref[...] = _dot(h, w3[...]) + b3[...]


def _mlp3_body_bf16(x_ref, w1, b1, w2, b2, w3, b3, o_ref):
    h = _relu(_bdot(x_ref[...], w1[...]) + b1[...])
    h = _relu(_bdot(h, w2[...]) + b2[...])
    o_ref[...] = _bdot(h, w3[...]) + b3[...]


def _mlp3(x, params, bm, bf16=False):
    (w1, b1), (w2, b2), (w3, b3) = params
    m, k = x.shape
    grid = (m + bm - 1) // bm
    return pl.pallas_call(
        _mlp3_body_bf16 if bf16 else _mlp3_body,
        grid=(grid,),
        in_specs=[
            _row_spec(bm, k),
            _full_spec(w1.shape), _full_spec((1, b1.shape[0])),
            _full_spec(w2.shape), _full_spec((1, b2.shape[0])),
            _full_spec(w3.shape), _full_spec((1, b3.shape[0])),
        ],
        out_specs=_row_spec(bm, w3.shape[1]),
        out_shape=jax.ShapeDtypeStruct((m, w3.shape[1]), jnp.float32),
    )(x, w1, b1.reshape(1, -1), w2, b2.reshape(1, -1), w3, b3.reshape(1, -1))


def _proj_body(x_ref, w_ref, o_ref):
    o_ref[0] = _dot(x_ref[...], w_ref[0])


def _proj(node_lat, ws, wr, n_pad):
    bm = 640
    grid = (n_pad // bm, 2)
    w2 = jnp.stack([ws, wr])
    return pl.pallas_call(
        _proj_body,
        grid=grid,
        in_specs=[
            pl.BlockSpec((bm, 128), lambda i, j: (i, 0)),
            pl.BlockSpec((1, 128, 128), lambda i, j: (j, 0, 0)),
        ],
        out_specs=pl.BlockSpec((1, bm, 128), lambda i, j: (j, i, 0)),
        out_shape=jax.ShapeDtypeStruct((2, n_pad, 128), jnp.float32),
    )(node_lat, w2)


def _edge_body(gs_ref, gr_ref, el_ref, we, b1, w2, b2, w3, b3, o_ref):
    el = el_ref[...]
    h = _relu(gs_ref[0] + gr_ref[0] + _dot(el, we[...]) + b1[...])
    h = _relu(_dot(h, w2[...]) + b2[...])
    o_ref[...] = _dot(h, w3[...]) + b3[...] + el


def _edge_update(g, edge_lat, el_boff, we, b1, w2, b2, w3, b3):
    m = g.shape[1]
    grid = (m + BM_EDGE - 1) // BM_EDGE
    return pl.pallas_call(
        _edge_body,
        grid=(grid,),
        in_specs=[
            pl.BlockSpec((1, BM_EDGE, 128), lambda i: (0, i, 0)),
            pl.BlockSpec((1, BM_EDGE, 128), lambda i: (1, i, 0)),
            pl.BlockSpec((BM_EDGE, 128), lambda i: (i + el_boff, 0)),
            _full_spec((128, 128)), _full_spec((1, 128)),
            _full_spec((128, 128)), _full_spec((1, 128)),
            _full_spec((128, 128)), _full_spec((1, 128)),
        ],
        out_specs=_row_spec(BM_EDGE, 128),
        out_shape=jax.ShapeDtypeStruct((m, 128), jnp.float32),
    )(g, g, edge_lat, we, b1.reshape(1, -1), w2, b2.reshape(1, -1), w3, b3.reshape(1, -1))


def _node_body(nl_ref, pa0, pa1, pb0, pb1, wn, wa, b1, w2, b2, w3, b3, o_ref):
    nl = nl_ref[...]
    agg = (pa0[0] + pa1[0]) + (pb0[0] + pb1[0])
    h = _relu(_dot(nl, wn[...]) + _dot(agg, wa[...]) + b1[...])
    h = _relu(_dot(h, w2[...]) + b2[...])
    o_ref[...] = _dot(h, w3[...]) + b3[...] + nl


def _node_update(node_lat, parts_a, parts_b, wn, wa, b1, w2, b2, w3, b3):
    m = node_lat.shape[0]
    grid = (m + BM_NODE - 1) // BM_NODE
    p0 = pl.BlockSpec((1, BM_NODE, 128), lambda i: (0, i, 0))
    p1 = pl.BlockSpec((1, BM_NODE, 128), lambda i: (1, i, 0))
    return pl.pallas_call(
        _node_body,
        grid=(grid,),
        in_specs=[
            _row_spec(BM_NODE, 128), p0, p1, p0, p1,
            _full_spec((128, 128)), _full_spec((128, 128)), _full_spec((1, 128)),
            _full_spec((128, 128)), _full_spec((1, 128)),
            _full_spec((128, 128)), _full_spec((1, 128)),
        ],
        out_specs=_row_spec(BM_NODE, 128),
        out_shape=jax.ShapeDtypeStruct((m, 128), jnp.float32),
    )(node_lat, parts_a, parts_a, parts_b, parts_b, wn, wa, b1.reshape(1, -1),
      w2, b2.reshape(1, -1), w3, b3.reshape(1, -1))


def _sc_gather(tbl, idx, dep):
    per_s = idx.shape[3]
    e = per_s * NS
    ch = 136
    n_full = per_s // ch
    tail = per_s - n_full * ch
    n_pad = tbl.shape[1]
    rows_per_sub = n_pad // NS
    mesh = plsc.VectorSubcoreMesh(core_axis_name="c", subcore_axis_name="s",
                                  num_cores=NC, num_subcores=NS)

    @functools.partial(
        pl.kernel,
        out_type=jax.ShapeDtypeStruct((2, e, 128), jnp.float32),
        mesh=mesh,
        scratch_types=[
            pltpu.VMEM((per_s,), jnp.int32),
            pltpu.VMEM((ch, 128), jnp.float32),
            pltpu.VMEM((ch, 128), jnp.float32),
            pltpu.VMEM((64,), jnp.float32),
            pltpu.VMEM_SHARED((n_pad, 128), jnp.float32),
            [pltpu.SemaphoreType.DMA] * 2,
            [pltpu.SemaphoreType.DMA] * 2,
        ],
    )
    def k(tbl_hbm, idx_hbm, dep_hbm, out_hbm, eidx, rw0, rw1, depv, stbl, sg, sw):
        rows = (rw0, rw1)
        cid = lax.axis_index("c")
        sid = lax.axis_index("s")
        base = pl.multiple_of(sid * per_s, 8)
        srow = pl.multiple_of(sid * rows_per_sub, 8)

        pltpu.sync_copy(dep_hbm, depv)

        pltpu.sync_copy(tbl_hbm.at[cid, pl.ds(srow, rows_per_sub)],
                        stbl.at[pl.ds(srow, rows_per_sub)])
        pltpu.sync_copy(idx_hbm.at[cid, sid, 0], eidx)
        plsc.subcore_barrier()

        def g_start(i, b):
            ioff = pl.multiple_of(i * ch, 8)
            pltpu.async_copy(stbl.at[eidx.at[pl.ds(ioff, ch)]], rows[b], sg[b])

        def g_wait(b):
            pltpu.make_async_copy(tbl_hbm.at[0, pl.ds(0, ch)], rows[b], sg[b]).wait()

        def w_start(i, b):
            off = pl.multiple_of(base + i * ch, 8)
            pltpu.async_copy(rows[b], out_hbm.at[cid, pl.ds(off, ch)], sw[b])

        def w_wait(b):
            pltpu.make_async_copy(tbl_hbm.at[0, pl.ds(0, ch)], rows[b], sw[b]).wait()

        for b in range(2):
            g_start(jnp.int32(b), b)

        def body(g, _):
            for b in range(2):
                i = 2 * g + b
                g_wait(b)
                w_start(i, b)

                @pl.when(i + 2 < n_full)
                def _():
                    w_wait(b)
                    g_start(i + 2, b)

            return 0

        lax.fori_loop(0, n_full // 2, body, 0)

        if n_full % 2:
            bl = (n_full - 1) % 2
            g_wait(bl)
            off = pl.multiple_of(base + (n_full - 1) * ch, 8)
            pltpu.sync_copy(rows[bl], out_hbm.at[cid, pl.ds(off, ch)])
            w_wait((n_full - 2) % 2)
        else:
            w_wait((n_full - 2) % 2)
            w_wait((n_full - 1) % 2)

        if tail:
            toff = pl.multiple_of(jnp.int32(n_full * ch), 8)
            pltpu.sync_copy(stbl.at[eidx.at[pl.ds(toff, tail)]],
                            rows[0].at[pl.ds(0, tail)])
            pltpu.sync_copy(rows[0].at[pl.ds(0, tail)],
                            out_hbm.at[cid, pl.ds(base + toff, tail)])

    return k(tbl, idx, dep.reshape(-1)[:64])


def _sc_scatter(new_edge, receivers, n_pad, dep):
    e = new_edge.shape[0]
    per_w = e // NW
    ch = 104
    n_full = per_w // ch
    tail = per_w - n_full * ch
    rows_per_sub = n_pad // NS
    mesh = plsc.VectorSubcoreMesh(core_axis_name="c", subcore_axis_name="s",
                                  num_cores=NC, num_subcores=NS)
    zeros = jnp.zeros((n_pad, 128), jnp.float32)

    @functools.partial(
        pl.kernel,
        out_type=jax.ShapeDtypeStruct((NC, n_pad, 128), jnp.float32),
        mesh=mesh,
        scratch_types=[
            pltpu.VMEM((per_w,), jnp.int32),
            pltpu.VMEM((ch, 128), jnp.float32),
            pltpu.VMEM((ch, 128), jnp.float32),
            pltpu.VMEM((64,), jnp.float32),
            pltpu.VMEM_SHARED((n_pad, 128), jnp.float32),
            [pltpu.SemaphoreType.DMA] * 2,
        ],
    )
    def k(edge_hbm, r_hbm, z_hbm, dep_hbm, out_hbm, ridx, rw0, rw1, depv, acc, se):
        rows = (rw0, rw1)
        cid = lax.axis_index("c")
        sid = lax.axis_index("s")
        wid = sid * NC + cid
        base = pl.multiple_of(wid * per_w, 8)
        srow = pl.multiple_of(sid * rows_per_sub, 8)

        pltpu.sync_copy(dep_hbm, depv)
        pltpu.sync_copy(z_hbm.at[pl.ds(srow, rows_per_sub)],
                        acc.at[pl.ds(srow, rows_per_sub)])
        pltpu.sync_copy(r_hbm.at[pl.ds(base, per_w)], ridx)
        plsc.subcore_barrier()

        def l_start(i, b):
            off = pl.multiple_of(base + i * ch, 8)
            pltpu.async_copy(edge_hbm.at[pl.ds(off, ch)], rows[b], se[b])

        def l_wait(b):
            pltpu.make_async_copy(edge_hbm.at[pl.ds(0, ch)], rows[b], se[b]).wait()

        def sc_add(i, b):
            ioff = pl.multiple_of(i * ch, 8)
            pltpu.sync_copy(rows[b], acc.at[ridx.at[pl.ds(ioff, ch)]], add=True)

        for b in range(2):
            l_start(jnp.int32(b), b)

        def body(g, _):
            for b in range(2):
                i = 2 * g + b
                l_wait(b)
                sc_add(i, b)

                @pl.when(i + 2 < n_full)
                def _():
                    l_start(i + 2, b)

            return 0

        lax.fori_loop(0, (n_full // 2), body, 0)

        if n_full % 2:
            l_wait((n_full - 1) % 2)
            sc_add(jnp.int32(n_full - 1), (n_full - 1) % 2)

        if tail:
            toff = pl.multiple_of(base + n_full * ch, 8)
            pltpu.sync_copy(edge_hbm.at[pl.ds(toff, tail)],
                            rows[0].at[pl.ds(0, tail)])
            tioff = pl.multiple_of(jnp.int32(n_full * ch), 8)
            pltpu.sync_copy(rows[0].at[pl.ds(0, tail)],
                            acc.at[ridx.at[pl.ds(tioff, tail)]], add=True)

        plsc.subcore_barrier()
        pltpu.sync_copy(acc.at[pl.ds(srow, rows_per_sub)],
                        out_hbm.at[cid, pl.ds(srow, rows_per_sub)])

    return k(new_edge, receivers, zeros, dep.reshape(-1)[:64])


def kernel(node_features, edge_features, senders, receivers, enc_node, enc_edge, proc, dec):
    n_nodes = node_features.shape[0]
    e = senders.shape[0]
    eh = (e * 3 // 5) // 32000 * 32000
    n_pad = ((n_nodes + 639) // 640) * 640
    s_h = [senders[:eh], senders[eh:]]
    r_h = [receivers[:eh], receivers[eh:]]
    idx_h = [jnp.stack([s_h[h], r_h[h]]).reshape(2, NS, 1, -1) for h in range(2)]

    node_lat = _mlp3(node_features, enc_node, BM_NODE)
    el = [_mlp3(edge_features, enc_edge, BM_EDGE, bf16=True), None]
    el_boff = [0, eh // BM_EDGE]

    for edge_p, node_p in proc:
        (w1, b1), (w2, b2), (w3, b3) = edge_p
        ws, wr, we = w1[:128], w1[128:256], w1[256:]
        tbl = _proj(node_lat, ws, wr, n_pad)
        g0 = _sc_gather(tbl, idx_h[0], tbl)
        g1 = _sc_gather(tbl, idx_h[1], g0)
        ne0 = _edge_update(g0, el[0], el_boff[0], we, b1, w2, b2, w3, b3)
        p0 = _sc_scatter(ne0, r_h[0], n_pad, g1)
        ne1 = _edge_update(g1, el[1] if el[1] is not None else el[0],
                           el_boff[1], we, b1, w2, b2, w3, b3)
        p1 = _sc_scatter(ne1, r_h[1], n_pad, p0)
        (n1, nb1), (n2, nb2), (n3, nb3) = node_p
        node_lat = _node_update(node_lat, p0, p1, n1[:128], n1[128:], nb1,
                                n2, nb2, n3, nb3)
        el = [ne0, ne1]
        el_boff = [0, 0]

    return _mlp3(node_lat, dec, BM_NODE)


# speedup vs baseline: 1.0480x; 1.0480x over previous
import functools

import jax
import jax.numpy as jnp
from jax import lax
from jax.experimental import pallas as pl
from jax.experimental.pallas import tpu as pltpu
from jax.experimental.pallas import tpu_sc as plsc

NC, NS = 2, 16
NW = NC * NS
CHUNK = 200

BM_NODE = 1000
BM_EDGE = 2000


def _row_spec(bm, k):
    return pl.BlockSpec((bm, k), lambda i: (i, 0))


def _full_spec(shape):
    return pl.BlockSpec(shape, lambda i: tuple(0 for _ in shape))


def _relu(x):
    return jnp.maximum(x, 0.0)


def _dot(x, w):
    return jnp.dot(x, w, preferred_element_type=jnp.float32)


def _mlp3_body(x_ref, w1, b1, w2, b2, w3, b3, o_ref):
    h = _relu(_dot(x_ref[...], w1[...]) + b1[...])
    h = _relu(_dot(h, w2[...]) + b2[...])
    o_ref[...] = _dot(h, w3[...]) + b3[...]


def _mlp3(x, params, bm):
    (w1, b1), (w2, b2), (w3, b3) = params
    m, k = x.shape
    grid = (m + bm - 1) // bm
    return pl.pallas_call(
        _mlp3_body,
        grid=(grid,),
        in_specs=[
            _row_spec(bm, k),
            _full_spec(w1.shape), _full_spec((1, b1.shape[0])),
            _full_spec(w2.shape), _full_spec((1, b2.shape[0])),
            _full_spec(w3.shape), _full_spec((1, b3.shape[0])),
        ],
        out_specs=_row_spec(bm, w3.shape[1]),
        out_shape=jax.ShapeDtypeStruct((m, w3.shape[1]), jnp.float32),
    )(x, w1, b1.reshape(1, -1), w2, b2.reshape(1, -1), w3, b3.reshape(1, -1))


def _proj_body(x_ref, w_ref, o_ref):
    o_ref[0] = _dot(x_ref[...], w_ref[0])


def _proj(node_lat, ws, wr, n_pad):
    bm = 640
    grid = (n_pad // bm, 2)
    w2 = jnp.stack([ws, wr])
    return pl.pallas_call(
        _proj_body,
        grid=grid,
        in_specs=[
            pl.BlockSpec((bm, 128), lambda i, j: (i, 0)),
            pl.BlockSpec((1, 128, 128), lambda i, j: (j, 0, 0)),
        ],
        out_specs=pl.BlockSpec((1, bm, 128), lambda i, j: (j, i, 0)),
        out_shape=jax.ShapeDtypeStruct((2, n_pad, 128), jnp.float32),
    )(node_lat, w2)


def _edge_body(gs_ref, gr_ref, el_ref, we, b1, w2, b2, w3, b3, o_ref):
    el = el_ref[...]
    h = _relu(gs_ref[0] + gr_ref[0] + _dot(el, we[...]) + b1[...])
    h = _relu(_dot(h, w2[...]) + b2[...])
    o_ref[...] = _dot(h, w3[...]) + b3[...] + el


def _edge_update(g, edge_lat, we, b1, w2, b2, w3, b3):
    m = edge_lat.shape[0]
    grid = (m + BM_EDGE - 1) // BM_EDGE
    return pl.pallas_call(
        _edge_body,
        grid=(grid,),
        in_specs=[
            pl.BlockSpec((1, BM_EDGE, 128), lambda i: (0, i, 0)),
            pl.BlockSpec((1, BM_EDGE, 128), lambda i: (1, i, 0)),
            _row_spec(BM_EDGE, 128),
            _full_spec((128, 128)), _full_spec((1, 128)),
            _full_spec((128, 128)), _full_spec((1, 128)),
            _full_spec((128, 128)), _full_spec((1, 128)),
        ],
        out_specs=_row_spec(BM_EDGE, 128),
        out_shape=jax.ShapeDtypeStruct((m, 128), jnp.float32),
    )(g, g, edge_lat, we, b1.reshape(1, -1), w2, b2.reshape(1, -1), w3, b3.reshape(1, -1))


def _node_body(nl_ref, p0_ref, p1_ref, wn, wa, b1, w2, b2, w3, b3, o_ref):
    nl = nl_ref[...]
    agg = p0_ref[0] + p1_ref[0]
    h = _relu(_dot(nl, wn[...]) + _dot(agg, wa[...]) + b1[...])
    h = _relu(_dot(h, w2[...]) + b2[...])
    o_ref[...] = _dot(h, w3[...]) + b3[...] + nl


def _node_update(node_lat, parts, wn, wa, b1, w2, b2, w3, b3):
    m = node_lat.shape[0]
    grid = (m + BM_NODE - 1) // BM_NODE
    return pl.pallas_call(
        _node_body,
        grid=(grid,),
        in_specs=[
            _row_spec(BM_NODE, 128),
            pl.BlockSpec((1, BM_NODE, 128), lambda i: (0, i, 0)),
            pl.BlockSpec((1, BM_NODE, 128), lambda i: (1, i, 0)),
            _full_spec((128, 128)), _full_spec((128, 128)), _full_spec((1, 128)),
            _full_spec((128, 128)), _full_spec((1, 128)),
            _full_spec((128, 128)), _full_spec((1, 128)),
        ],
        out_specs=_row_spec(BM_NODE, 128),
        out_shape=jax.ShapeDtypeStruct((m, 128), jnp.float32),
    )(node_lat, parts, parts, wn, wa, b1.reshape(1, -1),
      w2, b2.reshape(1, -1), w3, b3.reshape(1, -1))


def _sc_gather(tbl, idx):
    per_s = idx.shape[3]
    e = per_s * NS
    ch = 136
    n_full = per_s // ch
    tail = per_s - n_full * ch
    n_pad = tbl.shape[1]
    rows_per_sub = n_pad // NS
    mesh = plsc.VectorSubcoreMesh(core_axis_name="c", subcore_axis_name="s",
                                  num_cores=NC, num_subcores=NS)

    @functools.partial(
        pl.kernel,
        out_type=jax.ShapeDtypeStruct((2, e, 128), jnp.float32),
        mesh=mesh,
        scratch_types=[
            pltpu.VMEM((per_s,), jnp.int32),
            pltpu.VMEM((ch, 128), jnp.float32),
            pltpu.VMEM((ch, 128), jnp.float32),
            pltpu.VMEM_SHARED((n_pad, 128), jnp.float32),
            [pltpu.SemaphoreType.DMA] * 2,
            [pltpu.SemaphoreType.DMA] * 2,
        ],
    )
    def k(tbl_hbm, idx_hbm, out_hbm, eidx, rw0, rw1, stbl, sg, sw):
        rows = (rw0, rw1)
        cid = lax.axis_index("c")
        sid = lax.axis_index("s")
        base = pl.multiple_of(sid * per_s, 8)
        srow = pl.multiple_of(sid * rows_per_sub, 8)

        pltpu.sync_copy(tbl_hbm.at[cid, pl.ds(srow, rows_per_sub)],
                        stbl.at[pl.ds(srow, rows_per_sub)])
        pltpu.sync_copy(idx_hbm.at[cid, sid, 0], eidx)
        plsc.subcore_barrier()

        def g_start(i, b):
            ioff = pl.multiple_of(i * ch, 8)
            pltpu.async_copy(stbl.at[eidx.at[pl.ds(ioff, ch)]], rows[b], sg[b])

        def g_wait(b):
            pltpu.make_async_copy(tbl_hbm.at[0, pl.ds(0, ch)], rows[b], sg[b]).wait()

        def w_start(i, b):
            off = pl.multiple_of(base + i * ch, 8)
            pltpu.async_copy(rows[b], out_hbm.at[cid, pl.ds(off, ch)], sw[b])

        def w_wait(b):
            pltpu.make_async_copy(tbl_hbm.at[0, pl.ds(0, ch)], rows[b], sw[b]).wait()

        for b in range(2):
            g_start(jnp.int32(b), b)

        def body(g, _):
            for b in range(2):
                i = 2 * g + b
                g_wait(b)
                w_start(i, b)

                @pl.when(i + 2 < n_full)
                def _():
                    w_wait(b)
                    g_start(i + 2, b)

            return 0

        lax.fori_loop(0, n_full // 2, body, 0)

        if n_full % 2:
            bl = (n_full - 1) % 2
            g_wait(bl)
            off = pl.multiple_of(base + (n_full - 1) * ch, 8)
            pltpu.sync_copy(rows[bl], out_hbm.at[cid, pl.ds(off, ch)])
            w_wait((n_full - 2) % 2)
        else:
            w_wait((n_full - 2) % 2)
            w_wait((n_full - 1) % 2)

        if tail:
            toff = pl.multiple_of(jnp.int32(n_full * ch), 8)
            pltpu.sync_copy(stbl.at[eidx.at[pl.ds(toff, tail)]],
                            rows[0].at[pl.ds(0, tail)])
            pltpu.sync_copy(rows[0].at[pl.ds(0, tail)],
                            out_hbm.at[cid, pl.ds(base + toff, tail)])

    return k(tbl, idx)


def _sc_scatter(new_edge, receivers, n_pad):
    e = new_edge.shape[0]
    per_w = e // NW
    ch = 104
    n_full = per_w // ch
    tail = per_w - n_full * ch
    rows_per_sub = n_pad // NS
    mesh = plsc.VectorSubcoreMesh(core_axis_name="c", subcore_axis_name="s",
                                  num_cores=NC, num_subcores=NS)
    zeros = jnp.zeros((n_pad, 128), jnp.float32)

    @functools.partial(
        pl.kernel,
        out_type=jax.ShapeDtypeStruct((NC, n_pad, 128), jnp.float32),
        mesh=mesh,
        scratch_types=[
            pltpu.VMEM((per_w,), jnp.int32),
            pltpu.VMEM((ch, 128), jnp.float32),
            pltpu.VMEM((ch, 128), jnp.float32),
            pltpu.VMEM_SHARED((n_pad, 128), jnp.float32),
            [pltpu.SemaphoreType.DMA] * 2,
        ],
    )
    def k(edge_hbm, r_hbm, z_hbm, out_hbm, ridx, rw0, rw1, acc, se):
        rows = (rw0, rw1)
        cid = lax.axis_index("c")
        sid = lax.axis_index("s")
        wid = sid * NC + cid
        base = pl.multiple_of(wid * per_w, 8)
        srow = pl.multiple_of(sid * rows_per_sub, 8)

        pltpu.sync_copy(z_hbm.at[pl.ds(srow, rows_per_sub)],
                        acc.at[pl.ds(srow, rows_per_sub)])
        pltpu.sync_copy(r_hbm.at[pl.ds(base, per_w)], ridx)
        plsc.subcore_barrier()

        def l_start(i, b):
            off = pl.multiple_of(base + i * ch, 8)
            pltpu.async_copy(edge_hbm.at[pl.ds(off, ch)], rows[b], se[b])

        def l_wait(b):
            pltpu.make_async_copy(edge_hbm.at[pl.ds(0, ch)], rows[b], se[b]).wait()

        def sc_add(i, b):
            ioff = pl.multiple_of(i * ch, 8)
            pltpu.sync_copy(rows[b], acc.at[ridx.at[pl.ds(ioff, ch)]], add=True)

        for b in range(2):
            l_start(jnp.int32(b), b)

        def body(g, _):
            for b in range(2):
                i = 2 * g + b
                l_wait(b)
                sc_add(i, b)

                @pl.when(i + 2 < n_full)
                def _():
                    l_start(i + 2, b)

            return 0

        lax.fori_loop(0, (n_full // 2), body, 0)

        if n_full % 2:
            l_wait((n_full - 1) % 2)
            sc_add(jnp.int32(n_full - 1), (n_full - 1) % 2)

        if tail:
            toff = pl.multiple_of(base + n_full * ch, 8)
            pltpu.sync_copy(edge_hbm.at[pl.ds(toff, tail)],
                            rows[0].at[pl.ds(0, tail)])
            tioff = pl.multiple_of(jnp.int32(n_full * ch), 8)
            pltpu.sync_copy(rows[0].at[pl.ds(0, tail)],
                            acc.at[ridx.at[pl.ds(tioff, tail)]], add=True)

        plsc.subcore_barrier()
        pltpu.sync_copy(acc.at[pl.ds(srow, rows_per_sub)],
                        out_hbm.at[cid, pl.ds(srow, rows_per_sub)])

    return k(new_edge, receivers, zeros)


def kernel(node_features, edge_features, senders, receivers, enc_node, enc_edge, proc, dec):
    n_nodes = node_features.shape[0]
    n_pad = ((n_nodes + 639) // 640) * 640
    idx = jnp.stack([senders, receivers]).reshape(2, NS, 1, -1)

    node_lat = _mlp3(node_features, enc_node, BM_NODE)
    edge_lat = _mlp3(edge_features, enc_edge, BM_EDGE)

    for edge_p, node_p in proc:
        (w1, b1), (w2, b2), (w3, b3) = edge_p
        ws, wr, we = w1[:128], w1[128:256], w1[256:]
        tbl = _proj(node_lat, ws, wr, n_pad)
        g = _sc_gather(tbl, idx)
        new_edge = _edge_update(g, edge_lat, we, b1, w2, b2, w3, b3)
        parts = _sc_scatter(new_edge, receivers, n_pad)
        (n1, nb1), (n2, nb2), (n3, nb3) = node_p
        node_lat = _node_update(node_lat, parts, n1[:128], n1[128:], nb1,
                                n2, nb2, n3, nb3)
        edge_lat = new_edge

    return _mlp3(node_lat, dec, BM_NODE)
